# baseline (speedup 1.0000x reference)
import jax
import jax.numpy as jnp
from jax import lax
from jax.experimental import pallas as pl

_B, _TX, _S = 16, 4, 2
_SLABS = _B * _TX * _S
_ROWS_IN = 1792
_ROWS_DATA = 1664
_ROWS_PIL = 128


def _demap_body(in_ref, data_ref, pil_ref):
    x = in_ref[0]

    data_ref[0, 0:256] = x[0:256]
    data_ref[0, 320:1344] = x[384:1408]
    data_ref[0, 1408:1664] = x[1536:1792]

    r64 = lax.broadcasted_iota(jnp.int32, (64, 128), 0)
    i128 = lax.broadcasted_iota(jnp.int32, (64, 128), 1)
    se = (i128 == 2 * r64).astype(jnp.float32)
    so = (i128 == 2 * r64 + 1).astype(jnp.float32)

    jj = lax.broadcasted_iota(jnp.int32, (128, 128), 0)
    ll = lax.broadcasted_iota(jnp.int32, (128, 128), 1)
    h = 8 * ((ll % 64) // 4) + ll % 4
    lo = (ll < 64).astype(jnp.float32)
    hi = (ll >= 64).astype(jnp.float32)
    re1 = (jj == h).astype(jnp.float32) * lo
    re2 = (jj == h).astype(jnp.float32) * hi
    ro1 = (jj == h + 4).astype(jnp.float32) * lo
    ro2 = (jj == h + 4).astype(jnp.float32) * hi

    def dot(a, b):
        return jnp.dot(a, b, preferred_element_type=jnp.float32,
                       precision=lax.Precision.HIGHEST)

    for src_row, pil_row, data_row in ((256, 0, 256), (1408, 64, 1344)):
        xs = x[src_row:src_row + 128]
        t1 = dot(se, xs)
        t2 = dot(so, xs)
        pil_ref[0, pil_row:pil_row + 64] = dot(t1, re1) + dot(t2, re2)
        data_ref[0, data_row:data_row + 64] = dot(t1, ro1) + dot(t2, ro2)


def _demap(x2):
    return pl.pallas_call(
        _demap_body,
        grid=(_SLABS,),
        in_specs=[pl.BlockSpec((1, _ROWS_IN, 128), lambda i: (i, 0, 0))],
        out_specs=[
            pl.BlockSpec((1, _ROWS_DATA, 128), lambda i: (i, 0, 0)),
            pl.BlockSpec((1, _ROWS_PIL, 128), lambda i: (i, 0, 0)),
        ],
        out_shape=[
            jax.ShapeDtypeStruct((_SLABS, _ROWS_DATA, 128), jnp.float32),
            jax.ShapeDtypeStruct((_SLABS, _ROWS_PIL, 128), jnp.float32),
        ],
    )(x2)


@jax.jit
def kernel(inputs):
    b, tx, s, re, n = inputs.shape
    x2 = inputs.reshape(_SLABS, _ROWS_IN, 128)
    data2, pil2 = _demap(x2)
    data = data2.reshape(b, tx, s, 53248, n)
    pilots = pil2.reshape(b, tx, s, 1024, n, n)
    return (data, pilots)
